# baseline (speedup 1.0000x reference)
import jax
import jax.numpy as jnp
from jax.experimental import pallas as pl
from jax.experimental.pallas import tpu as pltpu

_C = 200
_CPAD = 256


def _dot(a, b):
    return jnp.dot(a, b, preferred_element_type=jnp.float32)


def _dot_rhs_t(a, b):
    return jax.lax.dot_general(a, b, (((1,), (1,)), ((), ())),
                               preferred_element_type=jnp.float32)


def _scores(hb, Wa_ref, ba_ref, Wb_ref, bb_ref, WcT_ref, bcT_ref):
    gated = jnp.tanh(_dot(hb, Wa_ref[...]) + ba_ref[...]) * \
        jax.nn.sigmoid(_dot(hb, Wb_ref[...]) + bb_ref[...])
    return _dot_rhs_t(WcT_ref[...], gated.astype(jnp.bfloat16)) \
        + bcT_ref[...]


def _online(sc, m_ref, s_ref):
    m_old = m_ref[...]
    m_new = jnp.maximum(m_old, jnp.max(sc, axis=1, keepdims=True))
    m_ref[...] = m_new
    s_ref[...] = s_ref[...] * jnp.exp(m_old - m_new) + \
        jnp.sum(jnp.exp(sc - m_new), axis=1, keepdims=True)


def _stream_body(x1_ref, x2_ref, Wp_ref, bp_ref, Wvf_ref, bvf_ref,
                 Wva_ref, bva_ref, Wvb_ref, bvb_ref, WvcT_ref, bvcT_ref,
                 Wuf_ref, buf_ref, Wua_ref, bua_ref, Wub_ref, bub_ref,
                 WucT_ref, bucT_ref,
                 M1_ref, M2_ref,
                 hv_st, hu_st, sc_st,
                 mv_ref, sv_ref, mu_ref, su_ref):
    i = pl.program_id(0)
    n = pl.num_programs(0)
    G = n - 1
    bf16 = jnp.bfloat16

    @pl.when(i == 0)
    def _init():
        for r in (mv_ref, mu_ref):
            r[...] = jnp.full_like(r[...], -jnp.inf)
        for r in (sv_ref, su_ref):
            r[...] = jnp.zeros_like(r[...])

    @pl.when(i < G)
    def _stream():
        h1 = _dot(x1_ref[...].astype(bf16), Wp_ref[...]) + bp_ref[...]
        hvb = jnp.maximum(
            _dot(h1.astype(bf16), Wvf_ref[...]) + bvf_ref[...],
            0.0).astype(bf16)
        hv_st[pl.ds(i * _C, _C), :] = hvb
        scv = _scores(hvb, Wva_ref, bva_ref, Wvb_ref, bvb_ref,
                      WvcT_ref, bvcT_ref)
        sc_st[i, 0:2, 0:_C] = scv
        _online(scv, mv_ref, sv_ref)

        hub = jnp.maximum(
            _dot(x2_ref[...].astype(bf16), Wuf_ref[...]) + buf_ref[...],
            0.0).astype(bf16)
        hu_st[pl.ds(i * _C, _C), :] = hub
        scu = _scores(hub, Wua_ref, bua_ref, Wub_ref, bub_ref,
                      WucT_ref, bucT_ref)
        sc_st[i, 2:4, 0:_C] = scu
        _online(scu, mu_ref, su_ref)

    @pl.when(i == G)
    def _pool():
        mv, sv = mv_ref[...], sv_ref[...]
        mu, su = mu_ref[...], su_ref[...]
        dims = (((1,), (0,)), ((), ()))

        def body(g, accs):
            acc_v, acc_u = accs
            sc_g = sc_st[g]
            av = (jnp.exp(sc_g[0:2, 0:_C] - mv) / sv).astype(jnp.bfloat16)
            au = (jnp.exp(sc_g[2:4, 0:_C] - mu) / su).astype(jnp.bfloat16)
            hv_g = hv_st[pl.ds(g * _C, _C), :]
            hu_g = hu_st[pl.ds(g * _C, _C), :]
            acc_v = acc_v + jax.lax.dot_general(
                av, hv_g, dims, preferred_element_type=jnp.float32)
            acc_u = acc_u + jax.lax.dot_general(
                au, hu_g, dims, preferred_element_type=jnp.float32)
            return acc_v, acc_u

        z = jnp.zeros((2, 512), jnp.float32)
        acc_v, acc_u = jax.lax.fori_loop(0, G, body, (z, z))
        M1_ref[...] = acc_v
        M2_ref[...] = acc_u


def _tail_body(M1_ref, M2_ref,
               W1a_ref, b1a_ref, W1b_ref, b1b_ref,
               W3a_ref, b3a_ref, W3b_ref, b3b_ref,
               W2a_ref, b2a_ref, W2b_ref, b2b_ref,
               Wop_ref, bop_ref, Wg1_ref, bg1_ref, Wg2_ref, bg2_ref,
               Wc_ref, bc_ref, out_ref):
    relu = lambda v: jnp.maximum(v, 0.0)
    M1 = M1_ref[...]
    M2 = M2_ref[...]
    cat = jnp.concatenate([M1, M2], axis=1)
    e1 = relu(_dot(relu(_dot(M1, W1a_ref[...]) + b1a_ref[...]),
                   W1b_ref[...]) + b1b_ref[...])
    e3 = relu(_dot(relu(_dot(M2, W3a_ref[...]) + b3a_ref[...]),
                   W3b_ref[...]) + b3b_ref[...])
    z2 = relu(_dot(relu(_dot(cat, W2a_ref[...]) + b2a_ref[...]),
                   W2b_ref[...]) + b2b_ref[...])
    e2 = _dot(z2, Wop_ref[...]) + bop_ref[...]
    glog = _dot(relu(_dot(cat, Wg1_ref[...]) + bg1_ref[...]),
                Wg2_ref[...]) + bg2_ref[...]
    g = jax.nn.softmax(glog, axis=1)
    fused = g[:, 0:1] * e1 + g[:, 1:2] * e2 + g[:, 2:3] * e3
    logits = jnp.sum(fused * Wc_ref[...], axis=1, keepdims=True)
    out_ref[...] = logits.reshape(1, 2) + bc_ref[...]


def kernel(x1, x2, params):
    (Wp, bp, Wvf, bvf, Wva, bva, Wvb, bvb, Wvc, bvc,
     Wuf, buf, Wua, bua, Wub, bub, Wuc, buc,
     W1a, b1a, W1b, b1b, W3a, b3a, W3b, b3b,
     W2a, b2a, W2b, b2b, Wop, bop,
     Wg1, bg1, Wg2, bg2, Wc, bc) = params

    N = x1.shape[0]
    C = _C
    G = N // C
    f32 = jnp.float32
    bf16 = jnp.bfloat16

    row = lambda v: v.reshape(1, -1)
    w = lambda v: v.astype(bf16)
    const2 = lambda a: pl.BlockSpec(a.shape, lambda i: (0, 0))

    stream_in = [
        x1, x2, w(Wp), row(bp), w(Wvf), row(bvf),
        w(Wva), row(bva), w(Wvb), row(bvb), w(Wvc.T), bvc.reshape(2, 1),
        w(Wuf), row(buf), w(Wua), row(bua), w(Wub), row(bub),
        w(Wuc.T), buc.reshape(2, 1),
    ]
    in_specs = [
        pl.BlockSpec((C, x1.shape[1]), lambda i: (jnp.minimum(i, G - 1), 0)),
        pl.BlockSpec((C, x2.shape[1]), lambda i: (jnp.minimum(i, G - 1), 0)),
    ] + [const2(a) for a in stream_in[2:]]

    M1, M2 = pl.pallas_call(
        _stream_body,
        grid=(G + 1,),
        in_specs=in_specs,
        out_specs=[const2(jnp.zeros((2, 512))) for _ in range(2)],
        out_shape=[jax.ShapeDtypeStruct((2, 512), f32) for _ in range(2)],
        scratch_shapes=[
            pltpu.VMEM((N, 512), bf16),
            pltpu.VMEM((N, 512), bf16),
            pltpu.VMEM((G, 8, _CPAD), f32),
            pltpu.VMEM((2, 1), f32), pltpu.VMEM((2, 1), f32),
            pltpu.VMEM((2, 1), f32), pltpu.VMEM((2, 1), f32),
        ],
        compiler_params=pltpu.CompilerParams(
            dimension_semantics=("arbitrary",)),
    )(*stream_in)

    tail_in = [
        M1, M2, W1a, row(b1a), W1b, row(b1b),
        W3a, row(b3a), W3b, row(b3b),
        W2a, row(b2a), W2b, row(b2b), Wop, row(bop),
        Wg1, row(bg1), Wg2, row(bg2), Wc, row(bc),
    ]
    out = pl.pallas_call(
        _tail_body,
        out_shape=jax.ShapeDtypeStruct((1, 2), f32),
    )(*tail_in)
    return out


# speedup vs baseline: 1.2835x; 1.2835x over previous
import jax
import jax.numpy as jnp
from jax.experimental import pallas as pl
from jax.experimental.pallas import tpu as pltpu

_C = 1000
_CPAD = 1024
_CP = 2000


def _dot(a, b):
    return jnp.dot(a, b, preferred_element_type=jnp.float32)


def _dot_rhs_t(a, b):
    return jax.lax.dot_general(a, b, (((1,), (1,)), ((), ())),
                               preferred_element_type=jnp.float32)


def _scores(hb, Wa_ref, ba_ref, Wb_ref, bb_ref, WcT_ref, bcT_ref):
    gated = jnp.tanh(_dot(hb, Wa_ref[...]) + ba_ref[...]) * \
        jax.nn.sigmoid(_dot(hb, Wb_ref[...]) + bb_ref[...])
    return _dot_rhs_t(WcT_ref[...], gated.astype(jnp.bfloat16)) \
        + bcT_ref[...]


def _online(sc, m_ref, s_ref):
    m_old = m_ref[...]
    m_new = jnp.maximum(m_old, jnp.max(sc, axis=1, keepdims=True))
    m_ref[...] = m_new
    s_ref[...] = s_ref[...] * jnp.exp(m_old - m_new) + \
        jnp.sum(jnp.exp(sc - m_new), axis=1, keepdims=True)


def _stream_body(x1_ref, x2_ref, Wp_ref, bp_ref, Wvf_ref, bvf_ref,
                 Wva_ref, bva_ref, Wvb_ref, bvb_ref, WvcT_ref, bvcT_ref,
                 Wuf_ref, buf_ref, Wua_ref, bua_ref, Wub_ref, bub_ref,
                 WucT_ref, bucT_ref,
                 hv_ref, hu_ref, sc_ref, mv_out, sv_out, mu_out, su_out,
                 mv_ref, sv_ref, mu_ref, su_ref):
    i = pl.program_id(0)
    n = pl.num_programs(0)
    bf16 = jnp.bfloat16

    @pl.when(i == 0)
    def _init():
        for r in (mv_ref, mu_ref):
            r[...] = jnp.full_like(r[...], -jnp.inf)
        for r in (sv_ref, su_ref):
            r[...] = jnp.zeros_like(r[...])

    h1 = _dot(x1_ref[...].astype(bf16), Wp_ref[...]) + bp_ref[...]
    hvb = jnp.maximum(
        _dot(h1.astype(bf16), Wvf_ref[...]) + bvf_ref[...], 0.0).astype(bf16)
    hv_ref[...] = hvb
    scv = _scores(hvb, Wva_ref, bva_ref, Wvb_ref, bvb_ref, WvcT_ref, bvcT_ref)
    sc_ref[0, 0:2, 0:_C] = scv
    _online(scv, mv_ref, sv_ref)

    hub = jnp.maximum(
        _dot(x2_ref[...].astype(bf16), Wuf_ref[...]) + buf_ref[...],
        0.0).astype(bf16)
    hu_ref[...] = hub
    scu = _scores(hub, Wua_ref, bua_ref, Wub_ref, bub_ref, WucT_ref, bucT_ref)
    sc_ref[0, 2:4, 0:_C] = scu
    _online(scu, mu_ref, su_ref)

    @pl.when(i == n - 1)
    def _fin():
        mv_out[...] = mv_ref[...]
        sv_out[...] = sv_ref[...]
        mu_out[...] = mu_ref[...]
        su_out[...] = su_ref[...]


def _pool_body(hv_ref, hu_ref, sc_ref, mv_ref, sv_ref, mu_ref, su_ref,
               M1_ref, M2_ref, accv_ref, accu_ref):
    i = pl.program_id(0)
    n = pl.num_programs(0)
    bf16 = jnp.bfloat16

    @pl.when(i == 0)
    def _init():
        accv_ref[...] = jnp.zeros_like(accv_ref[...])
        accu_ref[...] = jnp.zeros_like(accu_ref[...])

    mv, sv = mv_ref[...], sv_ref[...]
    mu, su = mu_ref[...], su_ref[...]
    acc_v = accv_ref[...]
    acc_u = accu_ref[...]
    for k in range(_CP // _C):
        av = (jnp.exp(sc_ref[k, 0:2, 0:_C] - mv) / sv).astype(bf16)
        au = (jnp.exp(sc_ref[k, 2:4, 0:_C] - mu) / su).astype(bf16)
        acc_v = acc_v + _dot(av, hv_ref[pl.ds(k * _C, _C), :])
        acc_u = acc_u + _dot(au, hu_ref[pl.ds(k * _C, _C), :])
    accv_ref[...] = acc_v
    accu_ref[...] = acc_u

    @pl.when(i == n - 1)
    def _fin():
        M1_ref[...] = acc_v
        M2_ref[...] = acc_u


def _tail_body(M1_ref, M2_ref,
               W1a_ref, b1a_ref, W1b_ref, b1b_ref,
               W3a_ref, b3a_ref, W3b_ref, b3b_ref,
               W2a_ref, b2a_ref, W2b_ref, b2b_ref,
               Wop_ref, bop_ref, Wg1_ref, bg1_ref, Wg2_ref, bg2_ref,
               Wc_ref, bc_ref, out_ref):
    relu = lambda v: jnp.maximum(v, 0.0)
    td = lambda a, b: _dot(a.astype(jnp.bfloat16), b)
    M1 = M1_ref[...]
    M2 = M2_ref[...]
    cat = jnp.concatenate([M1, M2], axis=1)
    e1 = relu(td(relu(td(M1, W1a_ref[...]) + b1a_ref[...]),
                 W1b_ref[...]) + b1b_ref[...])
    e3 = relu(td(relu(td(M2, W3a_ref[...]) + b3a_ref[...]),
                 W3b_ref[...]) + b3b_ref[...])
    z2 = relu(td(relu(td(cat, W2a_ref[...]) + b2a_ref[...]),
                 W2b_ref[...]) + b2b_ref[...])
    e2 = td(z2, Wop_ref[...]) + bop_ref[...]
    glog = td(relu(td(cat, Wg1_ref[...]) + bg1_ref[...]),
              Wg2_ref[...]) + bg2_ref[...]
    g = jax.nn.softmax(glog, axis=1)
    fused = g[:, 0:1] * e1 + g[:, 1:2] * e2 + g[:, 2:3] * e3
    logits = jnp.sum(fused * Wc_ref[...], axis=1, keepdims=True)
    out_ref[...] = logits.reshape(1, 2) + bc_ref[...]


def kernel(x1, x2, params):
    (Wp, bp, Wvf, bvf, Wva, bva, Wvb, bvb, Wvc, bvc,
     Wuf, buf, Wua, bua, Wub, bub, Wuc, buc,
     W1a, b1a, W1b, b1b, W3a, b3a, W3b, b3b,
     W2a, b2a, W2b, b2b, Wop, bop,
     Wg1, bg1, Wg2, bg2, Wc, bc) = params

    N = x1.shape[0]
    C = _C
    G = N // C
    f32 = jnp.float32
    bf16 = jnp.bfloat16

    row = lambda v: v.reshape(1, -1)
    w = lambda v: v.astype(bf16)
    const2 = lambda a: pl.BlockSpec(a.shape, lambda i: (0, 0))

    stream_in = [
        x1, x2, w(Wp), row(bp), w(Wvf), row(bvf),
        w(Wva), row(bva), w(Wvb), row(bvb), w(Wvc.T), bvc.reshape(2, 1),
        w(Wuf), row(buf), w(Wua), row(bua), w(Wub), row(bub),
        w(Wuc.T), buc.reshape(2, 1),
    ]
    in_specs = [
        pl.BlockSpec((C, x1.shape[1]), lambda i: (i, 0)),
        pl.BlockSpec((C, x2.shape[1]), lambda i: (i, 0)),
    ] + [const2(a) for a in stream_in[2:]]

    ms_spec = pl.BlockSpec((2, 1), lambda i: (0, 0))
    hvb, hub, sc, mv, sv, mu, su = pl.pallas_call(
        _stream_body,
        grid=(G,),
        in_specs=in_specs,
        out_specs=[
            pl.BlockSpec((C, 512), lambda i: (i, 0)),
            pl.BlockSpec((C, 512), lambda i: (i, 0)),
            pl.BlockSpec((1, 8, _CPAD), lambda i: (i, 0, 0)),
            ms_spec, ms_spec, ms_spec, ms_spec,
        ],
        out_shape=[
            jax.ShapeDtypeStruct((N, 512), bf16),
            jax.ShapeDtypeStruct((N, 512), bf16),
            jax.ShapeDtypeStruct((G, 8, _CPAD), f32),
            jax.ShapeDtypeStruct((2, 1), f32),
            jax.ShapeDtypeStruct((2, 1), f32),
            jax.ShapeDtypeStruct((2, 1), f32),
            jax.ShapeDtypeStruct((2, 1), f32),
        ],
        scratch_shapes=[pltpu.VMEM((2, 1), f32) for _ in range(4)],
        compiler_params=pltpu.CompilerParams(
            dimension_semantics=("arbitrary",)),
    )(*stream_in)

    k = _CP // _C
    M1, M2 = pl.pallas_call(
        _pool_body,
        grid=(N // _CP,),
        in_specs=[
            pl.BlockSpec((_CP, 512), lambda i: (i, 0)),
            pl.BlockSpec((_CP, 512), lambda i: (i, 0)),
            pl.BlockSpec((k, 8, _CPAD), lambda i: (i, 0, 0)),
            ms_spec, ms_spec, ms_spec, ms_spec,
        ],
        out_specs=[const2(jnp.zeros((2, 512))) for _ in range(2)],
        out_shape=[jax.ShapeDtypeStruct((2, 512), f32) for _ in range(2)],
        scratch_shapes=[pltpu.VMEM((2, 512), f32) for _ in range(2)],
        compiler_params=pltpu.CompilerParams(
            dimension_semantics=("arbitrary",)),
    )(hvb, hub, sc, mv, sv, mu, su)

    tail_in = [
        M1, M2, w(W1a), row(b1a), w(W1b), row(b1b),
        w(W3a), row(b3a), w(W3b), row(b3b),
        w(W2a), row(b2a), w(W2b), row(b2b), w(Wop), row(bop),
        w(Wg1), row(bg1), w(Wg2), row(bg2), Wc, row(bc),
    ]
    out = pl.pallas_call(
        _tail_body,
        out_shape=jax.ShapeDtypeStruct((1, 2), f32),
    )(*tail_in)
    return out


# speedup vs baseline: 1.2938x; 1.0080x over previous
import jax
import jax.numpy as jnp
from jax.experimental import pallas as pl
from jax.experimental.pallas import tpu as pltpu

_C = 1000
_CPAD = 1024
_CP = 4000


def _dot(a, b):
    return jnp.dot(a, b, preferred_element_type=jnp.float32)


def _dot_rhs_t(a, b):
    return jax.lax.dot_general(a, b, (((1,), (1,)), ((), ())),
                               preferred_element_type=jnp.float32)


def _scores(hb, Wa_ref, ba_ref, Wb_ref, bb_ref, WcT_ref, bcT_ref):
    gated = jnp.tanh(_dot(hb, Wa_ref[...]) + ba_ref[...]) * \
        jax.nn.sigmoid(_dot(hb, Wb_ref[...]) + bb_ref[...])
    return _dot_rhs_t(WcT_ref[...], gated.astype(jnp.bfloat16)) \
        + bcT_ref[...]


def _online(sc, m_ref, s_ref):
    m_old = m_ref[...]
    m_new = jnp.maximum(m_old, jnp.max(sc, axis=1, keepdims=True))
    m_ref[...] = m_new
    s_ref[...] = s_ref[...] * jnp.exp(m_old - m_new) + \
        jnp.sum(jnp.exp(sc - m_new), axis=1, keepdims=True)


def _stream_body(x1_ref, x2_ref, Wp_ref, bp_ref, Wvf_ref, bvf_ref,
                 Wva_ref, bva_ref, Wvb_ref, bvb_ref, WvcT_ref, bvcT_ref,
                 Wuf_ref, buf_ref, Wua_ref, bua_ref, Wub_ref, bub_ref,
                 WucT_ref, bucT_ref,
                 h_ref, sc_ref, mv_out, sv_out, mu_out, su_out,
                 mv_ref, sv_ref, mu_ref, su_ref):
    i = pl.program_id(0)
    n = pl.num_programs(0)
    bf16 = jnp.bfloat16

    @pl.when(i == 0)
    def _init():
        for r in (mv_ref, mu_ref):
            r[...] = jnp.full_like(r[...], -jnp.inf)
        for r in (sv_ref, su_ref):
            r[...] = jnp.zeros_like(r[...])

    h1 = _dot(x1_ref[...].astype(bf16), Wp_ref[...]) + bp_ref[...]
    hvb = jnp.maximum(
        _dot(h1.astype(bf16), Wvf_ref[...]) + bvf_ref[...], 0.0).astype(bf16)
    h_ref[:, 0:512] = hvb
    scv = _scores(hvb, Wva_ref, bva_ref, Wvb_ref, bvb_ref, WvcT_ref, bvcT_ref)
    sc_ref[0, 0:2, 0:_C] = scv
    _online(scv, mv_ref, sv_ref)

    hub = jnp.maximum(
        _dot(x2_ref[...].astype(bf16), Wuf_ref[...]) + buf_ref[...],
        0.0).astype(bf16)
    h_ref[:, 512:1024] = hub
    scu = _scores(hub, Wua_ref, bua_ref, Wub_ref, bub_ref, WucT_ref, bucT_ref)
    sc_ref[0, 2:4, 0:_C] = scu
    _online(scu, mu_ref, su_ref)

    @pl.when(i == n - 1)
    def _fin():
        mv_out[...] = mv_ref[...]
        sv_out[...] = sv_ref[...]
        mu_out[...] = mu_ref[...]
        su_out[...] = su_ref[...]


def _pool_body(h_ref, sc_ref, mv_ref, sv_ref, mu_ref, su_ref,
               M1_ref, M2_ref, acc_ref):
    i = pl.program_id(0)
    n = pl.num_programs(0)
    bf16 = jnp.bfloat16

    @pl.when(i == 0)
    def _init():
        acc_ref[...] = jnp.zeros_like(acc_ref[...])

    mv, sv = mv_ref[...], sv_ref[...]
    mu, su = mu_ref[...], su_ref[...]
    blocks = []
    for k in range(_CP // _C):
        av = (jnp.exp(sc_ref[k, 0:2, 0:_C] - mv) / sv).astype(bf16)
        au = (jnp.exp(sc_ref[k, 2:4, 0:_C] - mu) / su).astype(bf16)
        blocks.append(jnp.concatenate([av, au], axis=0))
    a_full = jnp.concatenate(blocks, axis=1)
    acc = acc_ref[...] + _dot(a_full, h_ref[...])
    acc_ref[...] = acc

    @pl.when(i == n - 1)
    def _fin():
        M1_ref[...] = acc[0:2, 0:512]
        M2_ref[...] = acc[2:4, 512:1024]


def _tail_body(M1_ref, M2_ref,
               W1a_ref, b1a_ref, W1b_ref, b1b_ref,
               W3a_ref, b3a_ref, W3b_ref, b3b_ref,
               W2a_ref, b2a_ref, W2b_ref, b2b_ref,
               Wop_ref, bop_ref, Wg1_ref, bg1_ref, Wg2_ref, bg2_ref,
               Wc_ref, bc_ref, out_ref):
    relu = lambda v: jnp.maximum(v, 0.0)
    td = lambda a, b: _dot(a.astype(jnp.bfloat16), b)
    M1 = M1_ref[...]
    M2 = M2_ref[...]
    cat = jnp.concatenate([M1, M2], axis=1)
    e1 = relu(td(relu(td(M1, W1a_ref[...]) + b1a_ref[...]),
                 W1b_ref[...]) + b1b_ref[...])
    e3 = relu(td(relu(td(M2, W3a_ref[...]) + b3a_ref[...]),
                 W3b_ref[...]) + b3b_ref[...])
    z2 = relu(td(relu(td(cat, W2a_ref[...]) + b2a_ref[...]),
                 W2b_ref[...]) + b2b_ref[...])
    e2 = td(z2, Wop_ref[...]) + bop_ref[...]
    glog = td(relu(td(cat, Wg1_ref[...]) + bg1_ref[...]),
              Wg2_ref[...]) + bg2_ref[...]
    g = jax.nn.softmax(glog, axis=1)
    fused = g[:, 0:1] * e1 + g[:, 1:2] * e2 + g[:, 2:3] * e3
    logits = jnp.sum(fused * Wc_ref[...], axis=1, keepdims=True)
    out_ref[...] = logits.reshape(1, 2) + bc_ref[...]


def kernel(x1, x2, params):
    (Wp, bp, Wvf, bvf, Wva, bva, Wvb, bvb, Wvc, bvc,
     Wuf, buf, Wua, bua, Wub, bub, Wuc, buc,
     W1a, b1a, W1b, b1b, W3a, b3a, W3b, b3b,
     W2a, b2a, W2b, b2b, Wop, bop,
     Wg1, bg1, Wg2, bg2, Wc, bc) = params

    N = x1.shape[0]
    C = _C
    G = N // C
    f32 = jnp.float32
    bf16 = jnp.bfloat16

    row = lambda v: v.reshape(1, -1)
    w = lambda v: v.astype(bf16)
    const2 = lambda a: pl.BlockSpec(a.shape, lambda i: (0, 0))

    stream_in = [
        x1, x2, w(Wp), row(bp), w(Wvf), row(bvf),
        w(Wva), row(bva), w(Wvb), row(bvb), w(Wvc.T), bvc.reshape(2, 1),
        w(Wuf), row(buf), w(Wua), row(bua), w(Wub), row(bub),
        w(Wuc.T), buc.reshape(2, 1),
    ]
    in_specs = [
        pl.BlockSpec((C, x1.shape[1]), lambda i: (i, 0)),
        pl.BlockSpec((C, x2.shape[1]), lambda i: (i, 0)),
    ] + [const2(a) for a in stream_in[2:]]

    ms_spec = pl.BlockSpec((2, 1), lambda i: (0, 0))
    h_st, sc, mv, sv, mu, su = pl.pallas_call(
        _stream_body,
        grid=(G,),
        in_specs=in_specs,
        out_specs=[
            pl.BlockSpec((C, 1024), lambda i: (i, 0)),
            pl.BlockSpec((1, 8, _CPAD), lambda i: (i, 0, 0)),
            ms_spec, ms_spec, ms_spec, ms_spec,
        ],
        out_shape=[
            jax.ShapeDtypeStruct((N, 1024), bf16),
            jax.ShapeDtypeStruct((G, 8, _CPAD), f32),
            jax.ShapeDtypeStruct((2, 1), f32),
            jax.ShapeDtypeStruct((2, 1), f32),
            jax.ShapeDtypeStruct((2, 1), f32),
            jax.ShapeDtypeStruct((2, 1), f32),
        ],
        scratch_shapes=[pltpu.VMEM((2, 1), f32) for _ in range(4)],
        compiler_params=pltpu.CompilerParams(
            dimension_semantics=("arbitrary",)),
    )(*stream_in)

    k = _CP // _C
    M1, M2 = pl.pallas_call(
        _pool_body,
        grid=(N // _CP,),
        in_specs=[
            pl.BlockSpec((_CP, 1024), lambda i: (i, 0)),
            pl.BlockSpec((k, 8, _CPAD), lambda i: (i, 0, 0)),
            ms_spec, ms_spec, ms_spec, ms_spec,
        ],
        out_specs=[const2(jnp.zeros((2, 512))) for _ in range(2)],
        out_shape=[jax.ShapeDtypeStruct((2, 512), f32) for _ in range(2)],
        scratch_shapes=[pltpu.VMEM((4, 1024), f32)],
        compiler_params=pltpu.CompilerParams(
            dimension_semantics=("arbitrary",)),
    )(h_st, sc, mv, sv, mu, su)

    tail_in = [
        M1, M2, w(W1a), row(b1a), w(W1b), row(b1b),
        w(W3a), row(b3a), w(W3b), row(b3b),
        w(W2a), row(b2a), w(W2b), row(b2b), w(Wop), row(bop),
        w(Wg1), row(bg1), w(Wg2), row(bg2), Wc, row(bc),
    ]
    out = pl.pallas_call(
        _tail_body,
        out_shape=jax.ShapeDtypeStruct((1, 2), f32),
    )(*tail_in)
    return out
